# baseline (speedup 1.0000x reference)
import functools

import jax
import jax.numpy as jnp
from jax import lax
from jax.experimental import pallas as pl
from jax.experimental.pallas import tpu as pltpu
from jax.experimental.pallas import tpu_sc as plsc

_VOCAB = 100000
_EMB = 64
_HID = 128
_B = 1024
_L = 200

_NC = 2
_NS = 16
_NW = _NC * _NS

_G = 4
_GI = _G * _L
_GCHUNKS = tuple((o, min(128, _GI - o)) for o in range(0, _GI, 128))

_B0 = 128
_B1 = _B - _B0


def _issue_group(table_hbm, idx_v, buf, base, sem):
    for off, n in _GCHUNKS:
        pltpu.async_copy(
            table_hbm.at[idx_v.at[pl.ds(base + off, n)]],
            buf.at[pl.ds(off, n)],
            sem,
        )


def _drain_group(table_hbm, idx_v, buf, base, sem):
    for off, n in _GCHUNKS:
        pltpu.make_async_copy(
            table_hbm.at[idx_v.at[pl.ds(base + off, n)]],
            buf.at[pl.ds(off, n)],
            sem,
        ).wait()


def _accum_group(buf, pool_v, out_hbm, wbase, g):
    scale = jnp.float32(1.0 / _L)
    for r in range(_G):
        def add_r(k, acc):
            return tuple(
                acc[c] + buf[r * _L + k, pl.ds(c * 16, 16)] for c in range(4)
            )
        z = jnp.zeros((16,), jnp.float32)
        acc = lax.fori_loop(0, _L, add_r, (z, z, z, z))
        for c in range(4):
            pool_v[r, pl.ds(c * 16, 16)] = acc[c] * scale
    pltpu.sync_copy(pool_v, out_hbm.at[pl.ds(wbase + g * _G, _G)])


def _pool_sc(bpw, ng, idx_hbm, table_hbm, out_hbm, idx_v, buf_a, buf_b,
             pool_v, sem_a, sem_b):
    wid = lax.axis_index("s") * _NC + lax.axis_index("c")
    wbase = wid * bpw
    pltpu.sync_copy(idx_hbm.at[pl.ds(wbase * _L, bpw * _L)], idx_v)

    _issue_group(table_hbm, idx_v, buf_a, 0, sem_a)

    def pair_body(p, carry):
        g0 = p * 2
        i0 = pl.multiple_of(g0 * _GI, 8)

        @pl.when(g0 + 1 < ng)
        def _():
            _issue_group(table_hbm, idx_v, buf_b, i0 + _GI, sem_b)
        _drain_group(table_hbm, idx_v, buf_a, i0, sem_a)
        _accum_group(buf_a, pool_v, out_hbm, wbase, g0)

        @pl.when(g0 + 2 < ng)
        def _():
            _issue_group(table_hbm, idx_v, buf_a, i0 + 2 * _GI, sem_a)
        _drain_group(table_hbm, idx_v, buf_b, i0 + _GI, sem_b)
        _accum_group(buf_b, pool_v, out_hbm, wbase, g0 + 1)
        return carry

    lax.fori_loop(0, ng // 2, pair_body, 0)

    if ng % 2 == 1:
        gl = ng - 1
        il = gl * _GI
        _drain_group(table_hbm, idx_v, buf_a, il, sem_a)
        _accum_group(buf_a, pool_v, out_hbm, wbase, gl)


def _pool(idx_flat, emb_table, nb):
    bpw = nb // _NW
    ng = bpw // _G
    mesh = plsc.VectorSubcoreMesh(core_axis_name="c", subcore_axis_name="s")
    f = pl.kernel(
        functools.partial(_pool_sc, bpw, ng),
        out_type=jax.ShapeDtypeStruct((nb, _EMB), jnp.float32),
        mesh=mesh,
        scratch_types=[
            pltpu.VMEM((bpw * _L,), jnp.int32),
            pltpu.VMEM((_GI, _EMB), jnp.float32),
            pltpu.VMEM((_GI, _EMB), jnp.float32),
            pltpu.VMEM((_G, _EMB), jnp.float32),
            pltpu.SemaphoreType.DMA,
            pltpu.SemaphoreType.DMA,
        ],
        compiler_params=pltpu.CompilerParams(use_tc_tiling_on_sc=False),
    )
    return f(idx_flat, emb_table)


def _mlp_head_tc(pooled_ref, w1_ref, b1_ref, w2_ref, b2_ref, out_ref):
    h = jnp.dot(pooled_ref[...], w1_ref[...],
                preferred_element_type=jnp.float32)
    h = jnp.maximum(h + b1_ref[...], 0.0)
    out_ref[...] = jnp.dot(h, w2_ref[...],
                           preferred_element_type=jnp.float32) + b2_ref[...]


def _mlp_tail_tc(pooled_ref, w1_ref, b1_ref, w2_ref, b2_ref, prev_ref,
                 out_ref):
    del prev_ref
    h = jnp.dot(pooled_ref[...], w1_ref[...],
                preferred_element_type=jnp.float32)
    h = jnp.maximum(h + b1_ref[...], 0.0)
    out_ref[...] = jnp.dot(h, w2_ref[...],
                           preferred_element_type=jnp.float32) + b2_ref[...]


_TN = 4096
_RB = 128


def _mlp_head(pooled0, W1, b1, W2, b2):
    nv = pl.cdiv(_VOCAB, _TN)
    return pl.pallas_call(
        _mlp_head_tc,
        grid=(nv,),
        in_specs=[
            pl.BlockSpec((_B0, _EMB), lambda i: (0, 0)),
            pl.BlockSpec((_EMB, _HID), lambda i: (0, 0)),
            pl.BlockSpec((1, _HID), lambda i: (0, 0)),
            pl.BlockSpec((_HID, _TN), lambda i: (0, i)),
            pl.BlockSpec((1, _TN), lambda i: (0, i)),
        ],
        out_specs=pl.BlockSpec((_B0, _TN), lambda i: (0, i)),
        out_shape=jax.ShapeDtypeStruct((_B, _VOCAB), jnp.float32),
        compiler_params=pltpu.CompilerParams(
            dimension_semantics=("arbitrary",),
        ),
    )(pooled0, W1, b1, W2, b2)


def _mlp_tail(pooled1, W1, b1, W2, b2, prev):
    nv = pl.cdiv(_VOCAB, _TN)
    nj = _B1 // _RB
    return pl.pallas_call(
        _mlp_tail_tc,
        grid=(nv, nj),
        in_specs=[
            pl.BlockSpec((_RB, _EMB), lambda i, j: (j, 0)),
            pl.BlockSpec((_EMB, _HID), lambda i, j: (0, 0)),
            pl.BlockSpec((1, _HID), lambda i, j: (0, 0)),
            pl.BlockSpec((_HID, _TN), lambda i, j: (0, i)),
            pl.BlockSpec((1, _TN), lambda i, j: (0, i)),
            pl.BlockSpec(memory_space=pl.ANY),
        ],
        out_specs=pl.BlockSpec((_RB, _TN), lambda i, j: (j + _B0 // _RB, i)),
        out_shape=jax.ShapeDtypeStruct((_B, _VOCAB), jnp.float32),
        input_output_aliases={5: 0},
        compiler_params=pltpu.CompilerParams(
            dimension_semantics=("arbitrary", "arbitrary"),
        ),
    )(pooled1, W1, b1, W2, b2, prev)


def kernel(inputs, emb_table, W1, b1, W2, b2):
    idx_flat = lax.optimization_barrier(inputs.reshape(_B * _L))
    pooled0 = _pool(idx_flat[: _B0 * _L], emb_table, _B0)
    pooled1 = _pool(idx_flat[_B0 * _L:], emb_table, _B1)
    b1r = b1.reshape(1, _HID)
    b2r = b2.reshape(1, _VOCAB)
    head = _mlp_head(pooled0, W1, b1r, W2, b2r)
    return _mlp_tail(pooled1, W1, b1r, W2, b2r, head)


# speedup vs baseline: 1.1682x; 1.1682x over previous
import functools

import jax
import jax.numpy as jnp
from jax import lax
from jax.experimental import pallas as pl
from jax.experimental.pallas import tpu as pltpu
from jax.experimental.pallas import tpu_sc as plsc

_VOCAB = 100000
_EMB = 64
_HID = 128
_B = 1024
_L = 200

_NC = 2
_NS = 16
_NW = _NC * _NS
_BPW = _B // _NW

_G = 4
_NG = _BPW // _G
_GI = _G * _L
_GCHUNKS = tuple((o, min(128, _GI - o)) for o in range(0, _GI, 128))


def _issue_group(table_hbm, idx_v, buf, base, sem):
    for off, n in _GCHUNKS:
        pltpu.async_copy(
            table_hbm.at[idx_v.at[pl.ds(base + off, n)]],
            buf.at[pl.ds(off, n)],
            sem,
        )


def _drain_group(table_hbm, idx_v, buf, base, sem):
    for off, n in _GCHUNKS:
        pltpu.make_async_copy(
            table_hbm.at[idx_v.at[pl.ds(base + off, n)]],
            buf.at[pl.ds(off, n)],
            sem,
        ).wait()


def _accum_group(buf, pool_v, out_hbm, wbase, g):
    scale = jnp.float32(1.0 / _L)
    for r in range(_G):
        def add_r(k, acc):
            return tuple(
                acc[c] + buf[r * _L + k, pl.ds(c * 16, 16)] for c in range(4)
            )
        z = jnp.zeros((16,), jnp.float32)
        acc = lax.fori_loop(0, _L, add_r, (z, z, z, z))
        for c in range(4):
            pool_v[r, pl.ds(c * 16, 16)] = acc[c] * scale
    pltpu.sync_copy(pool_v, out_hbm.at[pl.ds(wbase + g * _G, _G)])


def _pool_sc(idx_hbm, table_hbm, out_hbm, idx_v, buf_a, buf_b, pool_v,
             sem_a, sem_b):
    wid = lax.axis_index("s") * _NC + lax.axis_index("c")
    wbase = wid * _BPW
    pltpu.sync_copy(idx_hbm.at[pl.ds(wbase * _L, _BPW * _L)], idx_v)

    _issue_group(table_hbm, idx_v, buf_a, 0, sem_a)

    def pair_body(p, carry):
        g0 = p * 2
        i0 = pl.multiple_of(g0 * _GI, 8)

        @pl.when(g0 + 1 < _NG)
        def _():
            _issue_group(table_hbm, idx_v, buf_b, i0 + _GI, sem_b)
        _drain_group(table_hbm, idx_v, buf_a, i0, sem_a)
        _accum_group(buf_a, pool_v, out_hbm, wbase, g0)

        @pl.when(g0 + 2 < _NG)
        def _():
            _issue_group(table_hbm, idx_v, buf_a, i0 + 2 * _GI, sem_a)
        _drain_group(table_hbm, idx_v, buf_b, i0 + _GI, sem_b)
        _accum_group(buf_b, pool_v, out_hbm, wbase, g0 + 1)
        return carry

    lax.fori_loop(0, _NG // 2, pair_body, 0)


def _pool(inputs, emb_table):
    mesh = plsc.VectorSubcoreMesh(core_axis_name="c", subcore_axis_name="s")
    f = pl.kernel(
        _pool_sc,
        out_type=jax.ShapeDtypeStruct((_B, _EMB), jnp.float32),
        mesh=mesh,
        scratch_types=[
            pltpu.VMEM((_BPW * _L,), jnp.int32),
            pltpu.VMEM((_GI, _EMB), jnp.float32),
            pltpu.VMEM((_GI, _EMB), jnp.float32),
            pltpu.VMEM((_G, _EMB), jnp.float32),
            pltpu.SemaphoreType.DMA,
            pltpu.SemaphoreType.DMA,
        ],
        compiler_params=pltpu.CompilerParams(use_tc_tiling_on_sc=False),
    )
    idx_flat = lax.optimization_barrier(inputs.reshape(_B * _L))
    return f(idx_flat, emb_table)


def _mlp_tc(pooled_ref, w1_ref, b1_ref, w2_ref, b2_ref, out_ref):
    h = jnp.dot(pooled_ref[...], w1_ref[...],
                preferred_element_type=jnp.float32)
    h = jnp.maximum(h + b1_ref[...], 0.0)
    out_ref[...] = jnp.dot(h, w2_ref[...],
                           preferred_element_type=jnp.float32) + b2_ref[...]


_TN = 2048


def _mlp(pooled, W1, b1, W2, b2):
    nv = pl.cdiv(_VOCAB, _TN)
    return pl.pallas_call(
        _mlp_tc,
        grid=(nv,),
        in_specs=[
            pl.BlockSpec((_B, _EMB), lambda i: (0, 0)),
            pl.BlockSpec((_EMB, _HID), lambda i: (0, 0)),
            pl.BlockSpec((1, _HID), lambda i: (0, 0)),
            pl.BlockSpec((_HID, _TN), lambda i: (0, i)),
            pl.BlockSpec((1, _TN), lambda i: (0, i)),
        ],
        out_specs=pl.BlockSpec((_B, _TN), lambda i: (0, i)),
        out_shape=jax.ShapeDtypeStruct((_B, _VOCAB), jnp.float32),
        compiler_params=pltpu.CompilerParams(
            dimension_semantics=("arbitrary",),
        ),
    )(pooled, W1, b1.reshape(1, _HID), W2, b2.reshape(1, _VOCAB))


def kernel(inputs, emb_table, W1, b1, W2, b2):
    pooled = _pool(inputs, emb_table)
    return _mlp(pooled, W1, b1, W2, b2)
